# baseline (speedup 1.0000x reference)
import functools

import jax
import jax.numpy as jnp
from jax import lax
from jax.experimental import pallas as pl
from jax.experimental.pallas import tpu as pltpu
from jax.experimental.pallas import tpu_sc as plsc


def _build_emb_kernel(B, L, V, DE, DG, NC, NW, NL):
    D = DE + DG
    LP = 56
    LA = 48
    LR = L - LA
    DGA = 256
    DGB = DG - DGA
    TW = 128
    per_w = B // NW
    mesh = plsc.VectorSubcoreMesh(core_axis_name="c", subcore_axis_name="s")

    @functools.partial(
        pl.kernel,
        mesh=mesh,
        out_type=jax.ShapeDtypeStruct((B, L, D), jnp.float32),
        scratch_types=[
            pltpu.VMEM((LP,), jnp.int32),
            pltpu.VMEM((LA, D), jnp.float32),
            pltpu.VMEM((LA, TW), jnp.float32),
            pltpu.VMEM((8, DE), jnp.float32),
            pltpu.VMEM((8, DGA), jnp.float32),
            pltpu.VMEM((8, TW), jnp.float32),
            pltpu.VMEM((LR, D), jnp.float32),
            pltpu.SemaphoreType.DMA,
            pltpu.SemaphoreType.DMA,
            pltpu.SemaphoreType.DMA,
            pltpu.SemaphoreType.DMA,
            pltpu.SemaphoreType.DMA,
            pltpu.SemaphoreType.DMA,
        ],
        compiler_params=pltpu.CompilerParams(
            use_tc_tiling_on_sc=True, needs_layout_passes=False),
    )
    def emb_kernel(idx_hbm, elmo_hbm, glove_hbm, tail_hbm, out_hbm,
                   idx_v, fused_v, t48_v, e8_v, g8_v, t8_v, rem_v,
                   esem, gsem, tsem, esem8, gsem8, tsem8):
        wid = lax.axis_index("s") * NC + lax.axis_index("c")
        base_b = wid * per_w
        lane = lax.iota(jnp.int32, NL)

        def move(src, dst, n_el, src_cols, dst_cols, src_off, dst_off):
            for t in range(n_el // NL):
                e = t * NL + lane
                vals = plsc.load_gather(
                    src, [e // src_cols, e % src_cols + src_off])
                plsc.store_scatter(
                    dst, [e // dst_cols, e % dst_cols + dst_off], vals)

        def body(i, carry):
            b = base_b + i
            pltpu.sync_copy(idx_hbm.at[pl.ds(b * LP, LP)], idx_v)
            ref48 = idx_v.at[pl.ds(0, LA)]
            ec = pltpu.async_copy(
                elmo_hbm.at[ref48], fused_v.at[:, pl.ds(0, DE)], esem)
            gc = pltpu.async_copy(
                glove_hbm.at[ref48, pl.ds(0, DGA)],
                fused_v.at[:, pl.ds(DE, DGA)], gsem)
            tc = pltpu.async_copy(tail_hbm.at[ref48], t48_v, tsem)
            ref8 = idx_v.at[pl.ds(LA, 8)]
            ec8 = pltpu.async_copy(elmo_hbm.at[ref8], e8_v, esem)
            gc8 = pltpu.async_copy(
                glove_hbm.at[ref8, pl.ds(0, DGA)], g8_v, gsem)
            tc8 = pltpu.async_copy(tail_hbm.at[ref8], t8_v, tsem)
            tc.wait()
            for t in range(LA * DGB // NL):
                e = t * NL + lane
                vals = plsc.load_gather(t48_v, [e // DGB, e % DGB])
                plsc.store_scatter(
                    fused_v, [e // DGB, e % DGB + DE + DGA], vals)
            ec8.wait()
            move(e8_v, rem_v, LR * DE, DE, D, 0, 0)
            gc8.wait()
            move(g8_v, rem_v, LR * DGA, DGA, D, 0, DE)
            tc8.wait()
            for t in range(LR * DGB // NL + 1):
                e = t * NL + lane
                r = e // DGB
                cm = e % DGB
                vals = plsc.load_gather(t8_v, [r, cm])
                plsc.store_scatter(rem_v, [r, cm + DE + DGA], vals)
            ec.wait()
            gc.wait()
            pltpu.sync_copy(fused_v, out_hbm.at[b, pl.ds(0, LA)])
            pltpu.sync_copy(rem_v, out_hbm.at[b, pl.ds(LA, LR)])
            return carry

        lax.fori_loop(0, per_w, body, 0)

    return emb_kernel


def kernel(sentences, lengths, elmo_table, glove_table):
    B, L = sentences.shape
    V, DE = elmo_table.shape
    DG = glove_table.shape[1]

    info = plsc.get_sparse_core_info()
    NC, NS, NL = info.num_cores, info.num_subcores, info.num_lanes
    NW = NC * NS

    idx = jnp.pad(sentences.astype(jnp.int32), ((0, 0), (0, 56 - L)))
    idx = idx.reshape(B * 56)
    tail_t = jnp.pad(
        lax.slice(glove_table, (0, 256), (V, DG)), ((0, 0), (0, 128 - (DG - 256))))
    emb = _build_emb_kernel(B, L, V, DE, DG, NC, NW, NL)
    return emb(idx, elmo_table, glove_table, tail_t)
